# baseline (speedup 1.0000x reference)
import jax
import jax.numpy as jnp
from jax import lax
from jax.experimental import pallas as pl
from jax.experimental.pallas import tpu as pltpu
from jax.experimental.pallas import tpu_sc as plsc

_RADIUS2 = 0.1 * 0.1
_B, _M, _N, _K = 4, 2048, 8192, 2
_B_SC = 2
_B_TC = _B - _B_SC
_NC, _NS, _L = 2, 16, 16
_NW = _NC * _NS
_CPW = _B_SC * _M // _NW
_WPB = _M // _CPW
_NCHUNK = _N // _L
_W = 256
_NWIN = _N // _W
_NEG = -3.0e38
_TM = 1024


def _rne_bf16(v):
    u = lax.bitcast_convert_type(v, jnp.int32)
    r = (u + jnp.int32(0x7FFF) + ((u >> 16) & jnp.int32(1))) \
        & jnp.int32(-65536)
    return lax.bitcast_convert_type(r, jnp.float32)


def _sc_body(x_hbm, h_hbm, out0_hbm, out1_hbm,
             prx, pry, prz, pbx, pby, pbz, p2v,
             crx, cry, crz, cbx_v, cby_v, cbz_v, c2v,
             out0v, out1v):
    wid = lax.axis_index("s") * _NC + lax.axis_index("c")
    b = wid // _WPB
    m0 = (wid % _WPB) * _CPW

    pltpu.sync_copy(h_hbm.at[pl.ds((b * 3 + 0) * _N, _N)], prx)
    pltpu.sync_copy(h_hbm.at[pl.ds((b * 3 + 1) * _N, _N)], pry)
    pltpu.sync_copy(h_hbm.at[pl.ds((b * 3 + 2) * _N, _N)], prz)
    pltpu.sync_copy(x_hbm.at[pl.ds((b * 3 + 0) * _M + m0, _CPW)], crx)
    pltpu.sync_copy(x_hbm.at[pl.ds((b * 3 + 1) * _M + m0, _CPW)], cry)
    pltpu.sync_copy(x_hbm.at[pl.ds((b * 3 + 2) * _M + m0, _CPW)], crz)

    def prep_points(i, _):
        s = pl.ds(i * _L, _L)
        px, py, pz = prx[s], pry[s], prz[s]
        p2v[s] = (px * px + py * py) + pz * pz
        pbx[s] = _rne_bf16(px)
        pby[s] = _rne_bf16(py)
        pbz[s] = _rne_bf16(pz)
        return 0

    lax.fori_loop(0, _NCHUNK, prep_points, 0)

    def prep_centers(i, _):
        s = pl.ds(i * _L, _L)
        cx, cy, cz = crx[s], cry[s], crz[s]
        c2v[s] = (cx * cx + cy * cy) + cz * cz
        cbx_v[s] = _rne_bf16(cx)
        cby_v[s] = _rne_bf16(cy)
        cbz_v[s] = _rne_bf16(cz)
        return 0

    lax.fori_loop(0, _CPW // _L, prep_centers, 0)

    lane_iota = lax.iota(jnp.int32, _L)
    lane_off = [lane_iota + t * _L for t in range(_W // _L)]
    negf = jnp.float32(_NEG)

    def per_group(g, _):
        sgrp = pl.ds(g * _L, _L)
        cbx16, cby16, cbz16 = cbx_v[sgrp], cby_v[sgrp], cbz_v[sgrp]
        c216 = c2v[sgrp]

        def per_lane(k, carry):
            o0vec, o1vec = carry
            sel = lane_iota == k
            cbx = jnp.max(jnp.where(sel, cbx16, negf), axis=0)
            cby = jnp.max(jnp.where(sel, cby16, negf), axis=0)
            cbz = jnp.max(jnp.where(sel, cbz16, negf), axis=0)
            c2s = jnp.max(jnp.where(sel, c216, negf), axis=0)

            def cond(st):
                j, nf, _i0, _i1 = st
                return jnp.logical_and(j < _NWIN, nf < 2)

            def step(st):
                j, nf, i0, i1 = st
                base = j * _W
                rels = []
                mks = []
                for t in range(_W // _L):
                    s = pl.ds(base + t * _L, _L)
                    dot = (cbx * pbx[s] + cby * pby[s]) + cbz * pbz[s]
                    d2 = (c2s + p2v[s]) - 2.0 * dot
                    mks.append(d2 < _RADIUS2)
                for t in range(_W // _L):
                    lv = plsc.all_reduce_ffs(mks[t])
                    rels.append(jnp.where(lv < _L, lv + t * _L, _W))
                while len(rels) > 1:
                    rels = [jnp.minimum(rels[2 * i], rels[2 * i + 1])
                            for i in range(len(rels) // 2)]
                comb = rels[0]
                rels2 = []
                for t in range(_W // _L):
                    mk2 = jnp.logical_and(mks[t], lane_off[t] != comb)
                    lv2 = plsc.all_reduce_ffs(mk2)
                    rels2.append(jnp.where(lv2 < _L, lv2 + t * _L, _W))
                while len(rels2) > 1:
                    rels2 = [jnp.minimum(rels2[2 * i], rels2[2 * i + 1])
                             for i in range(len(rels2) // 2)]
                comb2 = rels2[0]
                r0 = comb[0]
                r1 = comb2[0]
                f0 = base + r0
                f1 = base + r1
                hit = r0 < _W
                both = r1 < _W
                ni0 = jnp.where(jnp.logical_and(hit, nf == 0), f0, i0)
                ni1 = jnp.where(hit, jnp.where(nf == 0, f1, f0), i1)
                nnf = jnp.where(
                    hit,
                    jnp.where(nf == 0,
                              jnp.where(both, jnp.int32(2), jnp.int32(1)),
                              jnp.int32(2)),
                    nf)
                return j + 1, nnf, ni0, ni1

            _j, nf, i0, i1 = lax.while_loop(
                cond, step,
                (jnp.int32(0), jnp.int32(0), jnp.int32(0), jnp.int32(0)))
            out0 = jnp.where(nf >= 1, i0, jnp.int32(0))
            out1 = jnp.where(nf >= 2, i1, out0)
            o0vec = jnp.where(sel, out0, o0vec)
            o1vec = jnp.where(sel, out1, o1vec)
            return o0vec, o1vec

        zeros = jnp.zeros((_L,), jnp.int32)
        o0vec, o1vec = lax.fori_loop(0, _L, per_lane, (zeros, zeros))
        out0v[sgrp] = o0vec
        out1v[sgrp] = o1vec
        return 0

    lax.fori_loop(0, _CPW // _L, per_group, 0)

    pltpu.sync_copy(out0v, out0_hbm.at[pl.ds(b * _M + m0, _CPW)])
    pltpu.sync_copy(out1v, out1_hbm.at[pl.ds(b * _M + m0, _CPW)])


def _tc_body(x_ref, h_ref, o_ref):
    c = x_ref[0]
    p = h_ref[0]
    n = p.shape[1]
    tm = c.shape[1]

    c2 = (c[0] * c[0] + c[1] * c[1]) + c[2] * c[2]
    p2 = (p[0] * p[0] + p[1] * p[1]) + p[2] * p[2]
    cb = c.astype(jnp.bfloat16)
    pb = p.astype(jnp.bfloat16)
    dot = lax.dot_general(cb, pb, (((0,), (0,)), ((), ())),
                          preferred_element_type=jnp.float32)
    dist2 = (c2[:, None] + p2[None, :]) - 2.0 * dot
    mask = dist2 < _RADIUS2

    sent = jnp.int32(n)
    iota = lax.broadcasted_iota(jnp.int32, (tm, n), 1)
    midx = jnp.where(mask, iota, sent)
    first = jnp.min(midx, axis=1)
    midx2 = jnp.where(midx == first[:, None], sent, midx)
    second = jnp.min(midx2, axis=1)

    out0 = jnp.where(first == sent, 0, first)
    out1 = jnp.where(second == sent, out0, second)
    o_ref[0] = jnp.stack([out0, out1], axis=1)


@jax.jit
def kernel(x, h):
    mesh = plsc.VectorSubcoreMesh(core_axis_name="c", subcore_axis_name="s")
    run_sc = pl.kernel(
        _sc_body,
        out_type=(jax.ShapeDtypeStruct((_B_SC * _M,), jnp.int32),
                  jax.ShapeDtypeStruct((_B_SC * _M,), jnp.int32)),
        mesh=mesh,
        compiler_params=pltpu.CompilerParams(needs_layout_passes=False),
        scratch_types=[
            pltpu.VMEM((_N,), jnp.float32),
            pltpu.VMEM((_N,), jnp.float32),
            pltpu.VMEM((_N,), jnp.float32),
            pltpu.VMEM((_N,), jnp.float32),
            pltpu.VMEM((_N,), jnp.float32),
            pltpu.VMEM((_N,), jnp.float32),
            pltpu.VMEM((_N,), jnp.float32),
            pltpu.VMEM((_CPW,), jnp.float32),
            pltpu.VMEM((_CPW,), jnp.float32),
            pltpu.VMEM((_CPW,), jnp.float32),
            pltpu.VMEM((_CPW,), jnp.float32),
            pltpu.VMEM((_CPW,), jnp.float32),
            pltpu.VMEM((_CPW,), jnp.float32),
            pltpu.VMEM((_CPW,), jnp.float32),
            pltpu.VMEM((_CPW,), jnp.int32),
            pltpu.VMEM((_CPW,), jnp.int32),
        ],
    )
    o0, o1 = run_sc(x[_B_TC:].reshape(-1), h[_B_TC:].reshape(-1))
    sc_out = jnp.stack([o0.reshape(_B_SC, _M), o1.reshape(_B_SC, _M)],
                       axis=-1)

    tc_out = pl.pallas_call(
        _tc_body,
        grid=(_B_TC, _M // _TM),
        in_specs=[
            pl.BlockSpec((1, 3, _TM), lambda i, j: (i, 0, j)),
            pl.BlockSpec((1, 3, _N), lambda i, j: (i, 0, 0)),
        ],
        out_specs=pl.BlockSpec((1, _TM, _K), lambda i, j: (i, j, 0)),
        out_shape=jax.ShapeDtypeStruct((_B_TC, _M, _K), jnp.int32),
    )(x[:_B_TC], h[:_B_TC])

    return jnp.concatenate([tc_out, sc_out], axis=0)


# speedup vs baseline: 1.1686x; 1.1686x over previous
import jax
import jax.numpy as jnp
from jax import lax
from jax.experimental import pallas as pl
from jax.experimental.pallas import tpu as pltpu
from jax.experimental.pallas import tpu_sc as plsc

_RADIUS2 = 0.1 * 0.1
_B, _M, _N, _K = 4, 2048, 8192, 2
_B_SC = 2
_B_TC = _B - _B_SC
_NC, _NS, _L = 2, 16, 16
_NW = _NC * _NS
_CPW = _B_SC * _M // _NW
_WPB = _M // _CPW
_NCHUNK = _N // _L
_W = 256
_NWIN = _N // _W
_NEG = -3.0e38
_TM = 1024


def _rne_bf16(v):
    u = lax.bitcast_convert_type(v, jnp.int32)
    r = (u + jnp.int32(0x7FFF) + ((u >> 16) & jnp.int32(1))) \
        & jnp.int32(-65536)
    return lax.bitcast_convert_type(r, jnp.float32)


def _sc_body(x_hbm, h_hbm, out0_hbm, out1_hbm,
             prx, pry, prz, pbx, pby, pbz, p2v,
             crx, cry, crz, cbx_v, cby_v, cbz_v, c2v,
             out0v, out1v):
    wid = lax.axis_index("s") * _NC + lax.axis_index("c")
    b = wid // _WPB
    m0 = (wid % _WPB) * _CPW

    pltpu.sync_copy(h_hbm.at[pl.ds((b * 3 + 0) * _N, _N)], prx)
    pltpu.sync_copy(h_hbm.at[pl.ds((b * 3 + 1) * _N, _N)], pry)
    pltpu.sync_copy(h_hbm.at[pl.ds((b * 3 + 2) * _N, _N)], prz)
    pltpu.sync_copy(x_hbm.at[pl.ds((b * 3 + 0) * _M + m0, _CPW)], crx)
    pltpu.sync_copy(x_hbm.at[pl.ds((b * 3 + 1) * _M + m0, _CPW)], cry)
    pltpu.sync_copy(x_hbm.at[pl.ds((b * 3 + 2) * _M + m0, _CPW)], crz)

    def prep_points(i, _):
        s = pl.ds(i * _L, _L)
        px, py, pz = prx[s], pry[s], prz[s]
        p2v[s] = (px * px + py * py) + pz * pz
        pbx[s] = _rne_bf16(px)
        pby[s] = _rne_bf16(py)
        pbz[s] = _rne_bf16(pz)
        return 0

    lax.fori_loop(0, _NCHUNK, prep_points, 0)

    def prep_centers(i, _):
        s = pl.ds(i * _L, _L)
        cx, cy, cz = crx[s], cry[s], crz[s]
        c2v[s] = (cx * cx + cy * cy) + cz * cz
        cbx_v[s] = _rne_bf16(cx)
        cby_v[s] = _rne_bf16(cy)
        cbz_v[s] = _rne_bf16(cz)
        return 0

    lax.fori_loop(0, _CPW // _L, prep_centers, 0)

    lane_iota = lax.iota(jnp.int32, _L)
    lane_off = [lane_iota + t * _L for t in range(_W // _L)]
    negf = jnp.float32(_NEG)

    def per_group(g, _):
        sgrp = pl.ds(g * _L, _L)
        cbx16, cby16, cbz16 = cbx_v[sgrp], cby_v[sgrp], cbz_v[sgrp]
        c216 = c2v[sgrp]

        def per_lane(k, carry):
            o0vec, o1vec = carry
            sel = lane_iota == k
            cbx = jnp.max(jnp.where(sel, cbx16, negf), axis=0)
            cby = jnp.max(jnp.where(sel, cby16, negf), axis=0)
            cbz = jnp.max(jnp.where(sel, cbz16, negf), axis=0)
            c2s = jnp.max(jnp.where(sel, c216, negf), axis=0)

            def cond(st):
                j, nf, _i0, _i1 = st
                return jnp.logical_and(j < _NWIN, nf < 2)

            def step(st):
                j, nf, i0, i1 = st
                base = j * _W
                rels = []
                mks = []
                for t in range(_W // _L):
                    s = pl.ds(base + t * _L, _L)
                    dot = (cbx * pbx[s] + cby * pby[s]) + cbz * pbz[s]
                    d2 = (c2s + p2v[s]) - 2.0 * dot
                    mks.append(d2 < _RADIUS2)
                for t in range(_W // _L):
                    lv = plsc.all_reduce_ffs(mks[t])
                    rels.append(jnp.where(lv < _L, lv + t * _L, _W))
                while len(rels) > 1:
                    rels = [jnp.minimum(rels[2 * i], rels[2 * i + 1])
                            for i in range(len(rels) // 2)]
                comb = rels[0]
                rels2 = []
                for t in range(_W // _L):
                    mk2 = jnp.logical_and(mks[t], lane_off[t] != comb)
                    lv2 = plsc.all_reduce_ffs(mk2)
                    rels2.append(jnp.where(lv2 < _L, lv2 + t * _L, _W))
                while len(rels2) > 1:
                    rels2 = [jnp.minimum(rels2[2 * i], rels2[2 * i + 1])
                             for i in range(len(rels2) // 2)]
                comb2 = rels2[0]
                r0 = comb[0]
                r1 = comb2[0]
                f0 = base + r0
                f1 = base + r1
                hit = r0 < _W
                both = r1 < _W
                ni0 = jnp.where(jnp.logical_and(hit, nf == 0), f0, i0)
                ni1 = jnp.where(hit, jnp.where(nf == 0, f1, f0), i1)
                nnf = jnp.where(
                    hit,
                    jnp.where(nf == 0,
                              jnp.where(both, jnp.int32(2), jnp.int32(1)),
                              jnp.int32(2)),
                    nf)
                return j + 1, nnf, ni0, ni1

            _j, nf, i0, i1 = lax.while_loop(
                cond, step,
                (jnp.int32(0), jnp.int32(0), jnp.int32(0), jnp.int32(0)))
            out0 = jnp.where(nf >= 1, i0, jnp.int32(0))
            out1 = jnp.where(nf >= 2, i1, out0)
            o0vec = jnp.where(sel, out0, o0vec)
            o1vec = jnp.where(sel, out1, o1vec)
            return o0vec, o1vec

        zeros = jnp.zeros((_L,), jnp.int32)
        o0vec, o1vec = lax.fori_loop(0, _L, per_lane, (zeros, zeros))
        out0v[sgrp] = o0vec
        out1v[sgrp] = o1vec
        return 0

    lax.fori_loop(0, _CPW // _L, per_group, 0)

    pltpu.sync_copy(out0v, out0_hbm.at[pl.ds(b * _M + m0, _CPW)])
    pltpu.sync_copy(out1v, out1_hbm.at[pl.ds(b * _M + m0, _CPW)])


def _tc_body(x_ref, h_ref, o_ref):
    c = x_ref[0]
    p = h_ref[0]
    n = p.shape[1]
    tm = c.shape[1]

    c2 = (c[0] * c[0] + c[1] * c[1]) + c[2] * c[2]
    p2 = (p[0] * p[0] + p[1] * p[1]) + p[2] * p[2]
    cb = c.astype(jnp.bfloat16)
    pb = p.astype(jnp.bfloat16)
    dot = lax.dot_general(cb, pb, (((0,), (0,)), ((), ())),
                          preferred_element_type=jnp.float32)
    dist2 = (c2[:, None] + p2[None, :]) - 2.0 * dot
    mask = dist2 < _RADIUS2

    sent = jnp.int32(n)
    iota = lax.broadcasted_iota(jnp.int32, (tm, n), 1)
    midx = jnp.where(mask, iota, sent)
    first = jnp.min(midx, axis=1)
    midx2 = jnp.where(midx == first[:, None], sent, midx)
    second = jnp.min(midx2, axis=1)

    out0 = jnp.where(first == sent, 0, first)
    out1 = jnp.where(second == sent, out0, second)
    o_ref[0] = jnp.stack([out0, out1], axis=0)


@jax.jit
def kernel(x, h):
    mesh = plsc.VectorSubcoreMesh(core_axis_name="c", subcore_axis_name="s")
    run_sc = pl.kernel(
        _sc_body,
        out_type=(jax.ShapeDtypeStruct((_B_SC * _M,), jnp.int32),
                  jax.ShapeDtypeStruct((_B_SC * _M,), jnp.int32)),
        mesh=mesh,
        compiler_params=pltpu.CompilerParams(needs_layout_passes=False),
        scratch_types=[
            pltpu.VMEM((_N,), jnp.float32),
            pltpu.VMEM((_N,), jnp.float32),
            pltpu.VMEM((_N,), jnp.float32),
            pltpu.VMEM((_N,), jnp.float32),
            pltpu.VMEM((_N,), jnp.float32),
            pltpu.VMEM((_N,), jnp.float32),
            pltpu.VMEM((_N,), jnp.float32),
            pltpu.VMEM((_CPW,), jnp.float32),
            pltpu.VMEM((_CPW,), jnp.float32),
            pltpu.VMEM((_CPW,), jnp.float32),
            pltpu.VMEM((_CPW,), jnp.float32),
            pltpu.VMEM((_CPW,), jnp.float32),
            pltpu.VMEM((_CPW,), jnp.float32),
            pltpu.VMEM((_CPW,), jnp.float32),
            pltpu.VMEM((_CPW,), jnp.int32),
            pltpu.VMEM((_CPW,), jnp.int32),
        ],
    )
    o0, o1 = run_sc(x[_B_TC:].reshape(-1), h[_B_TC:].reshape(-1))
    sc_out = jnp.stack([o0.reshape(_B_SC, _M), o1.reshape(_B_SC, _M)],
                       axis=-1)

    tc_out = pl.pallas_call(
        _tc_body,
        grid=(_B_TC, _M // _TM),
        in_specs=[
            pl.BlockSpec((1, 3, _TM), lambda i, j: (i, 0, j)),
            pl.BlockSpec((1, 3, _N), lambda i, j: (i, 0, 0)),
        ],
        out_specs=pl.BlockSpec((1, _K, _TM), lambda i, j: (i, 0, j)),
        out_shape=jax.ShapeDtypeStruct((_B_TC, _K, _M), jnp.int32),
    )(x[:_B_TC], h[:_B_TC])
    tc_out = jnp.transpose(tc_out, (0, 2, 1))

    return jnp.concatenate([tc_out, sc_out], axis=0)
